# Initial kernel scaffold; baseline (speedup 1.0000x reference)
import jax
import jax.numpy as jnp
from jax.experimental import pallas as pl


def kernel(x, mask, gamma, beta):
    raise NotImplementedError("write your pallas kernel here")


# speedup vs baseline: 1.1772x; 1.1772x over previous
import jax
import jax.numpy as jnp
from jax.experimental import pallas as pl
from jax.experimental.pallas import tpu as pltpu

D = 4096
EPS = 1e-5
ROWS_PER_BLOCK = 512


def _stats_body(x_ref, m_ref, sum_ref, sq_ref, cnt_ref):
    @pl.when(pl.program_id(0) == 0)
    def _init():
        sum_ref[...] = jnp.zeros_like(sum_ref)
        sq_ref[...] = jnp.zeros_like(sq_ref)
        cnt_ref[...] = jnp.zeros_like(cnt_ref)

    x = x_ref[...]
    m = m_ref[...]
    xm = x * m
    sum_ref[...] += jnp.sum(xm, axis=0, keepdims=True)
    sq_ref[...] += jnp.sum(xm * x, axis=0, keepdims=True)
    cnt_ref[...] += jnp.sum(m, axis=0, keepdims=True)


def _norm_body(x_ref, m_ref, sum_ref, sq_ref, cnt_ref, g_ref, b_ref, o_ref):
    n = jnp.maximum(cnt_ref[0, 0], 1.0)
    rn = 1.0 / n
    mean = sum_ref[...] * rn
    var = jnp.maximum(sq_ref[...] * rn - mean * mean, 0.0)
    inv = jax.lax.rsqrt(var + EPS)
    scale = inv * g_ref[...]
    shift = b_ref[...] - mean * scale
    x = x_ref[...]
    xn = x * scale + shift
    o_ref[...] = jnp.where(m_ref[...] > 0.0, xn, x)


def kernel(x, mask, gamma, beta):
    B, T, _D = x.shape
    N = B * T
    xf = x.reshape(N, D)
    mf = mask.reshape(N, 1).astype(jnp.float32)
    g2 = gamma.reshape(1, D)
    b2 = beta.reshape(1, D)

    R = ROWS_PER_BLOCK
    nblk = N // R

    sums, sqs, cnt = pl.pallas_call(
        _stats_body,
        grid=(nblk,),
        in_specs=[
            pl.BlockSpec((R, D), lambda i: (i, 0)),
            pl.BlockSpec((R, 1), lambda i: (i, 0)),
        ],
        out_specs=[
            pl.BlockSpec((1, D), lambda i: (0, 0)),
            pl.BlockSpec((1, D), lambda i: (0, 0)),
            pl.BlockSpec((1, 1), lambda i: (0, 0)),
        ],
        out_shape=[
            jax.ShapeDtypeStruct((1, D), jnp.float32),
            jax.ShapeDtypeStruct((1, D), jnp.float32),
            jax.ShapeDtypeStruct((1, 1), jnp.float32),
        ],
    )(xf, mf)

    out = pl.pallas_call(
        _norm_body,
        grid=(nblk,),
        in_specs=[
            pl.BlockSpec((R, D), lambda i: (i, 0)),
            pl.BlockSpec((R, 1), lambda i: (i, 0)),
            pl.BlockSpec((1, D), lambda i: (0, 0)),
            pl.BlockSpec((1, D), lambda i: (0, 0)),
            pl.BlockSpec((1, 1), lambda i: (0, 0)),
            pl.BlockSpec((1, D), lambda i: (0, 0)),
            pl.BlockSpec((1, D), lambda i: (0, 0)),
        ],
        out_specs=pl.BlockSpec((R, D), lambda i: (i, 0)),
        out_shape=jax.ShapeDtypeStruct((N, D), jnp.float32),
    )(xf, mf, sums, sqs, cnt, g2, b2)

    return out.reshape(B, T, D)
